# baseline (speedup 1.0000x reference)
import functools

import jax
import jax.numpy as jnp
from jax import lax
from jax.experimental import pallas as pl
from jax.experimental.pallas import tpu as pltpu
from jax.experimental.pallas import tpu_sc as plsc

EMB = 32
CHUNK = 128
GRP_CHUNKS = 2
GROUP = CHUNK * GRP_CHUNKS
NBUF = 4
NUM_WORKERS = 32
LANES = 16


@functools.cache
def _build(S, B):
    ng = (S * B) // GROUP
    assert ng % (NUM_WORKERS * NBUF) == 0 and B % GROUP == 0
    g_per_w = ng // NUM_WORKERS
    n_per_w = g_per_w * GROUP
    mesh = plsc.VectorSubcoreMesh(core_axis_name="c", subcore_axis_name="s")

    @functools.partial(
        pl.kernel,
        mesh=mesh,
        out_type=jax.ShapeDtypeStruct((S, EMB // 8, (B // 128) * 8 * 128), jnp.float32),
        scratch_types=[
            pltpu.VMEM((n_per_w,), jnp.int32),
            [pltpu.VMEM((GROUP, EMB), jnp.float32) for _ in range(NBUF)],
            [pltpu.VMEM((EMB // 8, GROUP * 8), jnp.float32) for _ in range(NBUF)],
            [pltpu.SemaphoreType.DMA for _ in range(NBUF)],
            [pltpu.SemaphoreType.DMA for _ in range(NBUF)],
        ],
        compiler_params=pltpu.CompilerParams(
            use_tc_tiling_on_sc=False, needs_layout_passes=False
        ),
    )
    def gather_kernel(table_hbm, idx_hbm, out_hbm, idx_v, gbufs, tbufs, gsems, osems):
        wid = lax.axis_index("s") * 2 + lax.axis_index("c")
        g0 = wid * g_per_w
        pltpu.sync_copy(idx_hbm.at[pl.ds(g0 * GROUP, n_per_w)], idx_v)

        nb = B // GROUP

        def g_start(g, b):
            for k in range(GRP_CHUNKS):
                pltpu.async_copy(
                    table_hbm.at[idx_v.at[pl.ds(g * GROUP + k * CHUNK, CHUNK)]],
                    gbufs[b].at[pl.ds(k * CHUNK, CHUNK)],
                    gsems[b],
                )

        def g_wait(b):
            for k in range(GRP_CHUNKS):
                pltpu.make_async_copy(
                    table_hbm.at[idx_v.at[pl.ds(0, CHUNK)]],
                    gbufs[b].at[pl.ds(k * CHUNK, CHUNK)],
                    gsems[b],
                ).wait()

        def o_start(gq, b):
            s = gq // nb
            b0 = (gq % nb) * (GROUP * 8)
            pltpu.async_copy(
                tbufs[b], out_hbm.at[s, :, pl.ds(b0, GROUP * 8)], osems[b]
            )

        def o_wait(b):
            pltpu.make_async_copy(
                tbufs[b], out_hbm.at[0, :, pl.ds(0, GROUP * 8)], osems[b]
            ).wait()

        def transpose(b):
            gbuf, tbuf = gbufs[b], tbufs[b]

            def tbody(grp, carry):
                lanes = lax.iota(jnp.int32, LANES)
                r = lanes + grp * LANES
                rmap = ((r >> 7) << 10) + (r & 127)
                for c in range(EMB):
                    diag = (lanes + c) & (EMB - 1)
                    vals = plsc.load_gather(gbuf, [r, diag])
                    plsc.store_scatter(
                        tbuf, [diag >> 3, rmap + ((diag & 7) << 7)], vals
                    )
                return carry

            lax.fori_loop(0, GROUP // LANES, tbody, 0)

        for b in range(NBUF):
            g_start(b, b)

        def body(i, carry):
            gg = i * NBUF
            for b in range(NBUF):
                g = gg + b
                g_wait(b)

                @pl.when(i > 0)
                def _():
                    o_wait(b)

                transpose(b)

                @pl.when(g + NBUF < g_per_w)
                def _():
                    g_start(g + NBUF, b)

                o_start(g0 + g, b)
            return carry

        lax.fori_loop(0, g_per_w // NBUF, body, 0)
        for b in range(NBUF):
            o_wait(b)

    return gather_kernel


def kernel(emb_table, indices):
    Bn, Sn = indices.shape
    idx_flat = indices.T.reshape(-1).astype(jnp.int32)
    out_t = _build(Sn, Bn)(emb_table, idx_flat)
    out5 = out_t.reshape(Sn, EMB // 8, Bn // 128, 8, 128)
    return out5.transpose(2, 4, 0, 1, 3).reshape(Bn, Sn, EMB)


# speedup vs baseline: 1.0368x; 1.0368x over previous
import functools

import jax
import jax.numpy as jnp
from jax import lax
from jax.experimental import pallas as pl
from jax.experimental.pallas import tpu as pltpu
from jax.experimental.pallas import tpu_sc as plsc

EMB = 32
CHUNK = 128
GRP_CHUNKS = 1
GROUP = CHUNK * GRP_CHUNKS
NBUF = 8
NUM_WORKERS = 32
LANES = 16


@functools.cache
def _build(S, B):
    ng = (S * B) // GROUP
    assert ng % (NUM_WORKERS * NBUF) == 0 and B % GROUP == 0
    g_per_w = ng // NUM_WORKERS
    n_per_w = g_per_w * GROUP
    mesh = plsc.VectorSubcoreMesh(core_axis_name="c", subcore_axis_name="s")

    @functools.partial(
        pl.kernel,
        mesh=mesh,
        out_type=jax.ShapeDtypeStruct((S, EMB // 8, (B // 128) * 8 * 128), jnp.float32),
        scratch_types=[
            pltpu.VMEM((n_per_w,), jnp.int32),
            [pltpu.VMEM((GROUP, EMB), jnp.float32) for _ in range(NBUF)],
            [pltpu.VMEM((GROUP * EMB,), jnp.float32) for _ in range(NBUF)],
            [pltpu.SemaphoreType.DMA for _ in range(NBUF)],
            [pltpu.SemaphoreType.DMA for _ in range(NBUF)],
        ],
        compiler_params=pltpu.CompilerParams(
            use_tc_tiling_on_sc=False, needs_layout_passes=False
        ),
    )
    def gather_kernel(table_hbm, idx_hbm, out_hbm, idx_v, gbufs, tbufs, gsems, osems):
        wid = lax.axis_index("s") * 2 + lax.axis_index("c")
        g0 = wid * g_per_w
        pltpu.sync_copy(idx_hbm.at[pl.ds(g0 * GROUP, n_per_w)], idx_v)

        nb = B // GROUP

        def g_start(g, b):
            for k in range(GRP_CHUNKS):
                pltpu.async_copy(
                    table_hbm.at[idx_v.at[pl.ds(g * GROUP + k * CHUNK, CHUNK)]],
                    gbufs[b].at[pl.ds(k * CHUNK, CHUNK)],
                    gsems[b],
                )

        def g_wait(b):
            for k in range(GRP_CHUNKS):
                pltpu.make_async_copy(
                    table_hbm.at[idx_v.at[pl.ds(0, CHUNK)]],
                    gbufs[b].at[pl.ds(k * CHUNK, CHUNK)],
                    gsems[b],
                ).wait()

        seg = GROUP * 8

        def o_start(gq, b):
            s = gq // nb
            b0 = (gq % nb) * seg
            for ch in range(EMB // 8):
                pltpu.async_copy(
                    tbufs[b].at[pl.ds(ch * seg, seg)],
                    out_hbm.at[s, ch, pl.ds(b0, seg)],
                    osems[b],
                )

        def o_wait(b):
            for ch in range(EMB // 8):
                pltpu.make_async_copy(
                    tbufs[b].at[pl.ds(ch * seg, seg)],
                    out_hbm.at[0, 0, pl.ds(0, seg)],
                    osems[b],
                ).wait()

        def transpose(b):
            gbuf, tbuf = gbufs[b], tbufs[b]

            def tbody(grp, carry):
                lanes = lax.iota(jnp.int32, LANES)
                r = lanes + grp * LANES
                base = ((lanes >> 3) << 10) + ((lanes & 7) << 7) + r
                for c in range(EMB):
                    kc = ((c >> 3) << 10) | ((c & 7) << 7)
                    vals = plsc.load_gather(gbuf, [r, lanes ^ c])
                    plsc.store_scatter(tbuf, [base ^ kc], vals)
                return carry

            lax.fori_loop(0, GROUP // LANES, tbody, 0)

        for b in range(NBUF):
            g_start(b, b)

        def body(i, carry):
            gg = i * NBUF
            for b in range(NBUF):
                g = gg + b
                g_wait(b)

                @pl.when(i > 0)
                def _():
                    o_wait(b)

                transpose(b)

                @pl.when(g + NBUF < g_per_w)
                def _():
                    g_start(g + NBUF, b)

                o_start(g0 + g, b)
            return carry

        lax.fori_loop(0, g_per_w // NBUF, body, 0)
        for b in range(NBUF):
            o_wait(b)

    return gather_kernel


def kernel(emb_table, indices):
    Bn, Sn = indices.shape
    idx_flat = indices.T.reshape(-1).astype(jnp.int32)
    out_t = _build(Sn, Bn)(emb_table, idx_flat)
    out5 = out_t.reshape(Sn, EMB // 8, Bn // 128, 8, 128)
    return out5.transpose(2, 4, 0, 1, 3).reshape(Bn, Sn, EMB)


# speedup vs baseline: 1.0530x; 1.0156x over previous
import functools

import jax
import jax.numpy as jnp
from jax import lax
from jax.experimental import pallas as pl
from jax.experimental.pallas import tpu as pltpu
from jax.experimental.pallas import tpu_sc as plsc

EMB = 32
CHUNK = 128
GRP_CHUNKS = 1
GROUP = CHUNK * GRP_CHUNKS
NBUF = 8
NUM_WORKERS = 32
LANES = 16


@functools.cache
def _build(S, B):
    ng = (S * B) // GROUP
    assert ng % (NUM_WORKERS * NBUF) == 0 and B % GROUP == 0
    g_per_w = ng // NUM_WORKERS
    n_per_w = g_per_w * GROUP
    mesh = plsc.VectorSubcoreMesh(core_axis_name="c", subcore_axis_name="s")

    @functools.partial(
        pl.kernel,
        mesh=mesh,
        out_type=jax.ShapeDtypeStruct((S, EMB // 8, (B // 128) * 8 * 128), jnp.float32),
        scratch_types=[
            pltpu.VMEM((n_per_w,), jnp.int32),
            [pltpu.VMEM((GROUP, EMB), jnp.float32) for _ in range(NBUF)],
            [pltpu.VMEM((EMB // 8, GROUP * 8), jnp.float32) for _ in range(NBUF)],
            [pltpu.SemaphoreType.DMA for _ in range(NBUF)],
            [pltpu.SemaphoreType.DMA for _ in range(NBUF)],
        ],
        compiler_params=pltpu.CompilerParams(
            use_tc_tiling_on_sc=False, needs_layout_passes=False
        ),
    )
    def gather_kernel(table_hbm, idx_hbm, out_hbm, idx_v, gbufs, tbufs, gsems, osems):
        wid = lax.axis_index("s") * 2 + lax.axis_index("c")
        g0 = wid * g_per_w
        pltpu.sync_copy(idx_hbm.at[pl.ds(g0 * GROUP, n_per_w)], idx_v)

        nb = B // GROUP

        def g_start(g, b):
            for k in range(GRP_CHUNKS):
                pltpu.async_copy(
                    table_hbm.at[idx_v.at[pl.ds(g * GROUP + k * CHUNK, CHUNK)]],
                    gbufs[b].at[pl.ds(k * CHUNK, CHUNK)],
                    gsems[b],
                )

        def g_wait(b):
            for k in range(GRP_CHUNKS):
                pltpu.make_async_copy(
                    table_hbm.at[idx_v.at[pl.ds(0, CHUNK)]],
                    gbufs[b].at[pl.ds(k * CHUNK, CHUNK)],
                    gsems[b],
                ).wait()

        seg = GROUP * 8

        def o_start(gq, b):
            s = gq // nb
            b0 = (gq % nb) * seg
            pltpu.async_copy(
                tbufs[b], out_hbm.at[s, :, pl.ds(b0, seg)], osems[b]
            )

        def o_wait(b):
            pltpu.make_async_copy(
                tbufs[b], out_hbm.at[0, :, pl.ds(0, seg)], osems[b]
            ).wait()

        def transpose(b):
            gbuf, tbuf = gbufs[b], tbufs[b]

            def tbody(grp, carry):
                lanes = lax.iota(jnp.int32, LANES)
                r = lanes + grp * LANES
                chl = lanes >> 3
                low = ((lanes & 7) << 7) + r
                for c in range(EMB):
                    vals = plsc.load_gather(gbuf, [r, lanes ^ c])
                    plsc.store_scatter(
                        tbuf, [chl ^ (c >> 3), low ^ ((c & 7) << 7)], vals
                    )
                return carry

            lax.fori_loop(0, GROUP // LANES, tbody, 0)

        for b in range(NBUF):
            g_start(b, b)

        def body(i, carry):
            gg = i * NBUF
            for b in range(NBUF):
                g = gg + b
                g_wait(b)

                @pl.when(i > 0)
                def _():
                    o_wait(b)

                transpose(b)

                @pl.when(g + NBUF < g_per_w)
                def _():
                    g_start(g + NBUF, b)

                o_start(g0 + g, b)
            return carry

        lax.fori_loop(0, g_per_w // NBUF, body, 0)
        for b in range(NBUF):
            o_wait(b)

    return gather_kernel


def kernel(emb_table, indices):
    Bn, Sn = indices.shape
    idx_flat = indices.T.reshape(-1).astype(jnp.int32)
    out_t = _build(Sn, Bn)(emb_table, idx_flat)
    out5 = out_t.reshape(Sn, EMB // 8, Bn // 128, 8, 128)
    return out5.transpose(2, 4, 0, 1, 3).reshape(Bn, Sn, EMB)


# speedup vs baseline: 1.0748x; 1.0208x over previous
import functools

import jax
import jax.numpy as jnp
from jax import lax
from jax.experimental import pallas as pl
from jax.experimental.pallas import tpu as pltpu
from jax.experimental.pallas import tpu_sc as plsc

EMB = 32
CHUNK = 128
GRP_CHUNKS = 1
GROUP = CHUNK * GRP_CHUNKS
NBUF = 8
NUM_WORKERS = 32
LANES = 16


@functools.cache
def _build(S, B):
    ng = (S * B) // GROUP
    assert ng % (NUM_WORKERS * NBUF) == 0 and B % GROUP == 0
    g_per_w = ng // NUM_WORKERS
    n_per_w = g_per_w * GROUP
    mesh = plsc.VectorSubcoreMesh(core_axis_name="c", subcore_axis_name="s")

    @functools.partial(
        pl.kernel,
        mesh=mesh,
        out_type=jax.ShapeDtypeStruct((S, EMB // 8, (B // 128) * 8 * 128), jnp.float32),
        scratch_types=[
            pltpu.VMEM((n_per_w,), jnp.int32),
            [pltpu.VMEM((GROUP, EMB), jnp.float32) for _ in range(NBUF)],
            [pltpu.VMEM((EMB // 8, GROUP * 8), jnp.float32) for _ in range(NBUF)],
            [pltpu.SemaphoreType.DMA for _ in range(NBUF)],
            [pltpu.SemaphoreType.DMA for _ in range(NBUF)],
        ],
        compiler_params=pltpu.CompilerParams(
            use_tc_tiling_on_sc=False, needs_layout_passes=False
        ),
    )
    def gather_kernel(table_hbm, idx_hbm, out_hbm, idx_v, gbufs, tbufs, gsems, osems):
        wid = lax.axis_index("s") * 2 + lax.axis_index("c")
        g0 = wid * g_per_w
        pltpu.sync_copy(idx_hbm.at[pl.ds(g0 * GROUP, n_per_w)], idx_v)

        nb = B // GROUP

        def g_start(g, b):
            for k in range(GRP_CHUNKS):
                pltpu.async_copy(
                    table_hbm.at[idx_v.at[pl.ds(g * GROUP + k * CHUNK, CHUNK)]],
                    gbufs[b].at[pl.ds(k * CHUNK, CHUNK)],
                    gsems[b],
                )

        def g_wait(b):
            for k in range(GRP_CHUNKS):
                pltpu.make_async_copy(
                    table_hbm.at[idx_v.at[pl.ds(0, CHUNK)]],
                    gbufs[b].at[pl.ds(k * CHUNK, CHUNK)],
                    gsems[b],
                ).wait()

        seg = GROUP * 8

        def o_start(gq, b):
            s = gq // nb
            b0 = (gq % nb) * seg
            pltpu.async_copy(
                tbufs[b], out_hbm.at[s, :, pl.ds(b0, seg)], osems[b]
            )

        def o_wait(b):
            pltpu.make_async_copy(
                tbufs[b], out_hbm.at[0, :, pl.ds(0, seg)], osems[b]
            ).wait()

        def transpose(b):
            gbuf, tbuf = gbufs[b], tbufs[b]

            def tbody(grp, carry):
                lanes = lax.iota(jnp.int32, LANES)
                r = lanes + grp * LANES
                rmap = ((r >> 7) << 10) + (r & 127)
                for c in range(EMB):
                    diag = (lanes + c) & (EMB - 1)
                    vals = plsc.load_gather(gbuf, [r, diag])
                    plsc.store_scatter(
                        tbuf, [diag >> 3, rmap + ((diag & 7) << 7)], vals
                    )
                return carry

            lax.fori_loop(0, GROUP // LANES, tbody, 0)

        for b in range(NBUF):
            g_start(b, b)

        def body(i, carry):
            gg = i * NBUF
            for b in range(NBUF):
                g = gg + b
                g_wait(b)

                @pl.when(i > 0)
                def _():
                    o_wait(b)

                transpose(b)

                @pl.when(g + NBUF < g_per_w)
                def _():
                    g_start(g + NBUF, b)

                o_start(g0 + g, b)
            return carry

        lax.fori_loop(0, g_per_w // NBUF, body, 0)
        for b in range(NBUF):
            o_wait(b)

    return gather_kernel


def kernel(emb_table, indices):
    Bn, Sn = indices.shape
    idx_flat = indices.T.reshape(-1).astype(jnp.int32)
    out_t = _build(Sn, Bn)(emb_table, idx_flat)
    out5 = out_t.reshape(Sn, EMB // 8, Bn // 128, 8, 128)
    return out5.transpose(2, 4, 0, 1, 3).reshape(Bn, Sn, EMB)
